# baseline (speedup 1.0000x reference)
import jax
import jax.numpy as jnp
from jax import lax
from jax.experimental import pallas as pl
from jax.experimental.pallas import tpu as pltpu
from jax.experimental.pallas import tpu_sc as plsc

NUM_CORES = 2
NUM_SUBCORES = 16
NW = NUM_CORES * NUM_SUBCORES
DIM = 64
BATCH = 16384
BPW = BATCH // NW
CH = 32
NCH = BPW // CH


def _body(h_hbm, r_hbm, t_hbm, ent_hbm, rel_hbm, out_hbm,
          hidx_v, ridx_v, tidx_v, hrow_v, rrow_v, trow_v, out_v, sem):
    cid = lax.axis_index("c")
    sid = lax.axis_index("s")
    wid = sid * NUM_CORES + cid
    base = wid * BPW

    pltpu.sync_copy(h_hbm.at[pl.ds(base, BPW)], hidx_v)
    pltpu.sync_copy(r_hbm.at[pl.ds(base, BPW)], ridx_v)
    pltpu.sync_copy(t_hbm.at[pl.ds(base, BPW)], tidx_v)

    lane = lax.iota(jnp.int32, 16)
    zero16 = jnp.zeros((16,), jnp.int32)

    def extract(ref, i):
        vec = ref[pl.ds((i // 16) * 16, 16)]
        return jnp.sum(jnp.where(lane == (i % 16), vec, zero16))

    def chunk(g, _):
        row0 = g * CH

        def fire(j, _):
            i = row0 + j
            pltpu.async_copy(ent_hbm.at[extract(hidx_v, i)], hrow_v.at[j], sem)
            pltpu.async_copy(rel_hbm.at[extract(ridx_v, i)], rrow_v.at[j], sem)
            pltpu.async_copy(ent_hbm.at[extract(tidx_v, i)], trow_v.at[j], sem)
            return 0

        lax.fori_loop(0, CH, fire, 0)
        pltpu.make_async_copy(ent_hbm.at[pl.ds(0, CH)], hrow_v, sem).wait()
        pltpu.make_async_copy(ent_hbm.at[pl.ds(0, CH)], rrow_v, sem).wait()
        pltpu.make_async_copy(ent_hbm.at[pl.ds(0, CH)], trow_v, sem).wait()

        def red(j, acc):
            s = jnp.zeros((16,), jnp.float32)
            for k in range(DIM // 16):
                sl = pl.ds(k * 16, 16)
                s = s + jnp.abs(hrow_v[j, sl] + rrow_v[j, sl] - trow_v[j, sl])
            tot = jnp.sum(s)
            acc = jnp.where(lane == (j % 16), -tot, acc)

            @pl.when(j % 16 == 15)
            def _():
                out_v[pl.ds(row0 + (j // 16) * 16, 16)] = acc

            return jnp.where(j % 16 == 15, jnp.zeros((16,), jnp.float32), acc)

        lax.fori_loop(0, CH, red, jnp.zeros((16,), jnp.float32))
        return 0

    lax.fori_loop(0, NCH, chunk, 0)

    pltpu.sync_copy(out_v, out_hbm.at[pl.ds(base, BPW)])


@jax.jit
def kernel(h, r, t, ent_weight, rel_weight):
    h1 = h.astype(jnp.int32)
    r1 = r.astype(jnp.int32)
    t1 = t.astype(jnp.int32)

    run = pl.kernel(
        _body,
        out_type=jax.ShapeDtypeStruct((BATCH,), jnp.float32),
        mesh=plsc.VectorSubcoreMesh(core_axis_name="c", subcore_axis_name="s"),
        compiler_params=pltpu.CompilerParams(needs_layout_passes=False),
        scratch_types=[
            pltpu.VMEM((BPW,), jnp.int32),
            pltpu.VMEM((BPW,), jnp.int32),
            pltpu.VMEM((BPW,), jnp.int32),
            pltpu.VMEM((CH, DIM), jnp.float32),
            pltpu.VMEM((CH, DIM), jnp.float32),
            pltpu.VMEM((CH, DIM), jnp.float32),
            pltpu.VMEM((BPW,), jnp.float32),
            pltpu.SemaphoreType.DMA,
        ],
    )
    return run(h1, r1, t1, ent_weight, rel_weight)


# speedup vs baseline: 1.0002x; 1.0002x over previous
import jax
import jax.numpy as jnp
from jax import lax
from jax.experimental import pallas as pl
from jax.experimental.pallas import tpu as pltpu
from jax.experimental.pallas import tpu_sc as plsc

NUM_CORES = 2
NUM_SUBCORES = 16
NW = NUM_CORES * NUM_SUBCORES
DIM = 64
BATCH = 16384
BPW = BATCH // NW
CH = 32
NCH = BPW // CH


def _body(h_hbm, r_hbm, t_hbm, ent_hbm, rel_hbm, out_hbm,
          hidx_v, ridx_v, tidx_v, hrow_v, rrow_v, trow_v, out_v, sem):
    cid = lax.axis_index("c")
    sid = lax.axis_index("s")
    wid = sid * NUM_CORES + cid
    base = wid * BPW

    pltpu.sync_copy(h_hbm.at[pl.ds(base, BPW)], hidx_v)
    pltpu.sync_copy(r_hbm.at[pl.ds(base, BPW)], ridx_v)
    pltpu.sync_copy(t_hbm.at[pl.ds(base, BPW)], tidx_v)

    lane = lax.iota(jnp.int32, 16)
    zero16 = jnp.zeros((16,), jnp.int32)

    def extract(ref, i):
        vec = ref[pl.ds((i // 16) * 16, 16)]
        return jnp.sum(jnp.where(lane == (i % 16), vec, zero16))

    def chunk(g, _):
        row0 = g * CH

        def fire(j, _):
            i = row0 + j
            pltpu.async_copy(ent_hbm.at[extract(hidx_v, i)], hrow_v.at[j], sem)
            pltpu.async_copy(rel_hbm.at[extract(ridx_v, i)], rrow_v.at[j], sem)
            pltpu.async_copy(ent_hbm.at[extract(tidx_v, i)], trow_v.at[j], sem)
            return 0

        lax.fori_loop(0, CH, fire, 0)
        pltpu.make_async_copy(ent_hbm.at[pl.ds(0, CH)], hrow_v, sem).wait()
        pltpu.make_async_copy(ent_hbm.at[pl.ds(0, CH)], rrow_v, sem).wait()
        pltpu.make_async_copy(ent_hbm.at[pl.ds(0, CH)], trow_v, sem).wait()

        def red(j, acc):
            s = jnp.zeros((16,), jnp.float32)
            for k in range(DIM // 16):
                sl = pl.ds(k * 16, 16)
                s = s + jnp.abs(hrow_v[j, sl] + rrow_v[j, sl] - trow_v[j, sl])
            tot = jnp.sum(s)
            acc = jnp.where(lane == (j % 16), -tot, acc)

            @pl.when(j % 16 == 15)
            def _():
                out_v[pl.ds(row0 + (j // 16) * 16, 16)] = acc

            return jnp.where(j % 16 == 15, jnp.zeros((16,), jnp.float32), acc)

        lax.fori_loop(0, CH, red, jnp.zeros((16,), jnp.float32))
        return 0

    lax.fori_loop(0, NCH, chunk, 0)

    pltpu.sync_copy(out_v, out_hbm.at[pl.ds(base, BPW)])


@jax.jit
def kernel(h, r, t, ent_weight, rel_weight):
    h1 = h.astype(jnp.int32)
    r1 = r.astype(jnp.int32)
    t1 = t.astype(jnp.int32)

    run = pl.kernel(
        _body,
        out_type=jax.ShapeDtypeStruct((BATCH,), jnp.float32),
        mesh=plsc.VectorSubcoreMesh(core_axis_name="c", subcore_axis_name="s"),
        compiler_params=pltpu.CompilerParams(
            needs_layout_passes=False, use_tc_tiling_on_sc=True),
        scratch_types=[
            pltpu.VMEM((BPW,), jnp.int32),
            pltpu.VMEM((BPW,), jnp.int32),
            pltpu.VMEM((BPW,), jnp.int32),
            pltpu.VMEM((CH, DIM), jnp.float32),
            pltpu.VMEM((CH, DIM), jnp.float32),
            pltpu.VMEM((CH, DIM), jnp.float32),
            pltpu.VMEM((BPW,), jnp.float32),
            pltpu.SemaphoreType.DMA,
        ],
    )
    return run(h1, r1, t1, ent_weight, rel_weight)
